# Initial kernel scaffold; baseline (speedup 1.0000x reference)
import jax
import jax.numpy as jnp
from jax.experimental import pallas as pl


def kernel(x, edge_index, Wl1, bl1, Wr1, br1, att1, bias1, g1, b1, Wl2, bl2, Wr2, br2, att2, bias2, g2, b2):
    raise NotImplementedError("write your pallas kernel here")


# speedup vs baseline: 20.0651x; 20.0651x over previous
import functools

import jax
import jax.numpy as jnp
from jax import lax
from jax.experimental import pallas as pl
from jax.experimental.pallas import tpu as pltpu
from jax.experimental.pallas import tpu_sc as plsc

NEG_SLOPE_ATT = 0.2
NEG_SLOPE_ACT = 0.01
BN_EPS = 1e-5
N, E = 10000, 640000
H = 4
S1 = (1.0 + NEG_SLOPE_ATT) / 2.0
S2 = (1.0 - NEG_SLOPE_ATT) / 2.0

NW = 32
EW = E // NW
CH = 80
NCH = EW // CH
NG = CH // 16
DR = H * N // 16

_BN = 1000

_SC_PARAMS = dict(use_tc_tiling_on_sc=False, needs_layout_passes=False)


def _proj(x, Wl, bl, Wr, br, attbd):
    n, k = x.shape
    m = Wl.shape[1]

    def body(x_ref, wl_ref, bl_ref, wr_ref, br_ref, abd_ref,
             xl_ref, xr_ref, al_ref, ar_ref):
        xv = x_ref[...]
        xl = jnp.dot(xv, wl_ref[...], preferred_element_type=jnp.float32) + bl_ref[...]
        xr = jnp.dot(xv, wr_ref[...], preferred_element_type=jnp.float32) + br_ref[...]
        xl_ref[...] = xl
        xr_ref[...] = xr
        al_ref[...] = jnp.dot(xl, abd_ref[...], preferred_element_type=jnp.float32)
        ar_ref[...] = jnp.dot(xr, abd_ref[...], preferred_element_type=jnp.float32)

    return pl.pallas_call(
        body,
        grid=(n // _BN,),
        in_specs=[
            pl.BlockSpec((_BN, k), lambda i: (i, 0)),
            pl.BlockSpec((k, m), lambda i: (0, 0)),
            pl.BlockSpec((1, m), lambda i: (0, 0)),
            pl.BlockSpec((k, m), lambda i: (0, 0)),
            pl.BlockSpec((1, m), lambda i: (0, 0)),
            pl.BlockSpec((m, H), lambda i: (0, 0)),
        ],
        out_specs=[
            pl.BlockSpec((_BN, m), lambda i: (i, 0)),
            pl.BlockSpec((_BN, m), lambda i: (i, 0)),
            pl.BlockSpec((_BN, H), lambda i: (i, 0)),
            pl.BlockSpec((_BN, H), lambda i: (i, 0)),
        ],
        out_shape=[
            jax.ShapeDtypeStruct((n, m), jnp.float32),
            jax.ShapeDtypeStruct((n, m), jnp.float32),
            jax.ShapeDtypeStruct((n, H), jnp.float32),
            jax.ShapeDtypeStruct((n, H), jnp.float32),
        ],
    )(x, Wl, bl.reshape(1, m), Wr, br.reshape(1, m), attbd)


def _combine_proj(outp, bias, g, b, Wl, bl, Wr, br, attbd):
    _, n, w = outp.shape
    m = Wl.shape[1]

    def body(o_ref, bias_ref, g_ref, b_ref, wl_ref, bl_ref, wr_ref, br_ref,
             abd_ref, xl_ref, xr_ref, al_ref, ar_ref):
        h = o_ref[0] + o_ref[1] + bias_ref[...]
        h = jnp.where(h > 0, h, NEG_SLOPE_ACT * h)
        h = g_ref[...] * h / jnp.sqrt(1.0 + BN_EPS) + b_ref[...]
        xl = jnp.dot(h, wl_ref[...], preferred_element_type=jnp.float32) + bl_ref[...]
        xr = jnp.dot(h, wr_ref[...], preferred_element_type=jnp.float32) + br_ref[...]
        xl_ref[...] = xl
        xr_ref[...] = xr
        al_ref[...] = jnp.dot(xl, abd_ref[...], preferred_element_type=jnp.float32)
        ar_ref[...] = jnp.dot(xr, abd_ref[...], preferred_element_type=jnp.float32)

    return pl.pallas_call(
        body,
        grid=(n // _BN,),
        in_specs=[
            pl.BlockSpec((2, _BN, w), lambda i: (0, i, 0)),
            pl.BlockSpec((1, w), lambda i: (0, 0)),
            pl.BlockSpec((1, w), lambda i: (0, 0)),
            pl.BlockSpec((1, w), lambda i: (0, 0)),
            pl.BlockSpec((w, m), lambda i: (0, 0)),
            pl.BlockSpec((1, m), lambda i: (0, 0)),
            pl.BlockSpec((w, m), lambda i: (0, 0)),
            pl.BlockSpec((1, m), lambda i: (0, 0)),
            pl.BlockSpec((m, H), lambda i: (0, 0)),
        ],
        out_specs=[
            pl.BlockSpec((_BN, m), lambda i: (i, 0)),
            pl.BlockSpec((_BN, m), lambda i: (i, 0)),
            pl.BlockSpec((_BN, H), lambda i: (i, 0)),
            pl.BlockSpec((_BN, H), lambda i: (i, 0)),
        ],
        out_shape=[
            jax.ShapeDtypeStruct((n, m), jnp.float32),
            jax.ShapeDtypeStruct((n, m), jnp.float32),
            jax.ShapeDtypeStruct((n, H), jnp.float32),
            jax.ShapeDtypeStruct((n, H), jnp.float32),
        ],
    )(outp, bias.reshape(1, w), g.reshape(1, w), b.reshape(1, w),
      Wl, bl.reshape(1, m), Wr, br.reshape(1, m), attbd)


def _combine_epi(outp, bias, g, b):
    _, n, w = outp.shape

    def body(o_ref, bias_ref, g_ref, b_ref, out_ref):
        h = o_ref[0] + o_ref[1] + bias_ref[...]
        h = jnp.where(h > 0, h, NEG_SLOPE_ACT * h)
        out_ref[...] = g_ref[...] * h / jnp.sqrt(1.0 + BN_EPS) + b_ref[...]

    return pl.pallas_call(
        body,
        grid=(n // _BN,),
        in_specs=[
            pl.BlockSpec((2, _BN, w), lambda i: (0, i, 0)),
            pl.BlockSpec((1, w), lambda i: (0, 0)),
            pl.BlockSpec((1, w), lambda i: (0, 0)),
            pl.BlockSpec((1, w), lambda i: (0, 0)),
        ],
        out_specs=pl.BlockSpec((_BN, w), lambda i: (i, 0)),
        out_shape=jax.ShapeDtypeStruct((n, w), jnp.float32),
    )(outp, bias.reshape(1, w), g.reshape(1, w), b.reshape(1, w))


def _rdenom(denp):
    def body(d_ref, out_ref):
        out_ref[...] = 1.0 / (d_ref[0] + d_ref[1] + 1e-16)

    return pl.pallas_call(
        body,
        in_specs=[pl.BlockSpec((2, DR, 16), lambda: (0, 0, 0))],
        out_specs=pl.BlockSpec((DR, 16), lambda: (0, 0)),
        out_shape=jax.ShapeDtypeStruct((DR, 16), jnp.float32),
        grid=(),
    )(denp)


def _mesh():
    return plsc.VectorSubcoreMesh(core_axis_name="c", subcore_axis_name="s")


def _logits_body(xlp, xrp, src_h, dst_h, atts2_h, log_h, wmax_h,
                 sidx, didx, xrow, yrow, logbuf, attbuf, wmaxbuf, sem1, sem2,
                 *, W, Wp, C):
    cid = lax.axis_index("c")
    sid = lax.axis_index("s")
    wid = sid * 2 + cid
    ebase = wid * EW
    pltpu.sync_copy(atts2_h, attbuf)
    iota = lax.iota(jnp.int32, 16)
    neg = jnp.full((16,), -3e38, jnp.float32)

    def chunk(k, mx):
        base = ebase + k * CH
        pltpu.sync_copy(src_h.at[pl.ds(base, CH)], sidx)
        pltpu.sync_copy(dst_h.at[pl.ds(base, CH)], didx)
        cp1 = pltpu.async_copy(xlp.at[sidx], xrow, sem1)
        cp2 = pltpu.async_copy(xrp.at[didx], yrow, sem2)
        cp1.wait()
        cp2.wait()
        mx = list(mx)
        e16s = [gi * 16 + iota for gi in range(NG)]
        for h in range(H):
            ch = jnp.full((16,), W + h, jnp.int32)
            accs = tuple(plsc.load_gather(xrow, [e16s[gi], ch])
                         + plsc.load_gather(yrow, [e16s[gi], ch])
                         for gi in range(NG))

            def cbody(c, accs):
                cs = jnp.full((16,), 0, jnp.int32) + c
                av = plsc.load_gather(attbuf, [cs])
                out = []
                for gi in range(NG):
                    z = (plsc.load_gather(xrow, [e16s[gi], cs])
                         + plsc.load_gather(yrow, [e16s[gi], cs]))
                    out.append(accs[gi] + jnp.abs(z) * av)
                return tuple(out)

            accs = lax.fori_loop(h * C, (h + 1) * C, cbody, accs, unroll=4)
            for gi in range(NG):
                plsc.store_scatter(logbuf, [e16s[gi] * H + h], accs[gi])
                mx[h] = jnp.maximum(mx[h], accs[gi])
        pltpu.sync_copy(logbuf, log_h.at[pl.ds(base * H, CH * H)])
        return tuple(mx)

    mx = lax.fori_loop(0, NCH, chunk, (neg, neg, neg, neg))
    for h in range(H):
        wmaxbuf[pl.ds(h * 16, 16)] = mx[h]
    pltpu.sync_copy(wmaxbuf, wmax_h.at[pl.ds(wid * (H * 16), H * 16)])


def _sc_logits(xlp, xrp, src, dst, atts2):
    Wp = xlp.shape[1]
    W = Wp - 16
    C = W // H
    body = functools.partial(_logits_body, W=W, Wp=Wp, C=C)
    return pl.kernel(
        body,
        out_type=[
            jax.ShapeDtypeStruct((E * H,), jnp.float32),
            jax.ShapeDtypeStruct((NW * H * 16,), jnp.float32),
        ],
        mesh=_mesh(),
        compiler_params=pltpu.CompilerParams(**_SC_PARAMS),
        scratch_types=[
            pltpu.VMEM((CH,), jnp.int32),
            pltpu.VMEM((CH,), jnp.int32),
            pltpu.VMEM((CH, Wp), jnp.float32),
            pltpu.VMEM((CH, Wp), jnp.float32),
            pltpu.VMEM((CH * H,), jnp.float32),
            pltpu.VMEM((W,), jnp.float32),
            pltpu.VMEM((H * 16,), jnp.float32),
            pltpu.SemaphoreType.DMA,
            pltpu.SemaphoreType.DMA,
        ],
    )(xlp, xrp, src, dst, atts2)


def _denom_body(log_h, dst_h, gmb_h, ex_h, denp_h,
                didx, lbuf, exbuf, gbuf, denloc, idxbuf, sdden):
    cid = lax.axis_index("c")
    sid = lax.axis_index("s")
    wid = sid * 2 + cid
    ebase = wid * EW
    iota = lax.iota(jnp.int32, 16)
    pltpu.sync_copy(gmb_h, gbuf)

    def zr(i, _):
        denloc[i, :] = jnp.zeros((16,), jnp.float32)
        return 0

    lax.fori_loop(0, DR, zr, 0)

    for t in range(20):
        for jj in range(8):
            st = min(jj * 16, 125 - 16)
            idxbuf[t, pl.ds(st, 16)] = t * 125 + st + iota

    @pl.when(sid == 0)
    def _():
        pltpu.sync_copy(denloc, sdden)

    def chunk(k, _):
        base = ebase + k * CH
        pltpu.sync_copy(dst_h.at[pl.ds(base, CH)], didx)
        pltpu.sync_copy(log_h.at[pl.ds(base * H, CH * H)], lbuf)
        for gi in range(NG):
            e16 = gi * 16 + iota
            d16 = didx[pl.ds(gi * 16, 16)]
            for h in range(H):
                l16 = plsc.load_gather(lbuf, [e16 * H + h])
                ev = jnp.exp(l16 - gbuf[pl.ds(h * 16, 16)])
                plsc.store_scatter(exbuf, [e16 * H + h], ev)
                f16 = d16 * H + h
                plsc.addupdate_scatter(
                    denloc,
                    [lax.shift_right_logical(f16, 4),
                     lax.bitwise_and(f16, 15)], ev)
        pltpu.sync_copy(exbuf, ex_h.at[pl.ds(base * H, CH * H)])
        return 0

    lax.fori_loop(0, NCH, chunk, 0)
    plsc.subcore_barrier()
    for t in range(20):
        pltpu.sync_copy(denloc.at[pl.ds(t * 125, 125)],
                        sdden.at[idxbuf.at[t]], add=True)
    plsc.subcore_barrier()

    @pl.when(sid == 0)
    def _():
        pltpu.sync_copy(sdden, denloc)
        pltpu.sync_copy(denloc, denp_h.at[cid])


def _sc_denom(logits, dst, gmb):
    return pl.kernel(
        _denom_body,
        out_type=[
            jax.ShapeDtypeStruct((E * H,), jnp.float32),
            jax.ShapeDtypeStruct((2, DR, 16), jnp.float32),
        ],
        mesh=_mesh(),
        compiler_params=pltpu.CompilerParams(**_SC_PARAMS),
        scratch_types=[
            pltpu.VMEM((CH,), jnp.int32),
            pltpu.VMEM((CH * H,), jnp.float32),
            pltpu.VMEM((CH * H,), jnp.float32),
            pltpu.VMEM((H * 16,), jnp.float32),
            pltpu.VMEM((DR, 16), jnp.float32),
            pltpu.VMEM((20, 125), jnp.int32),
            pltpu.VMEM_SHARED((DR, 16), jnp.float32),
        ],
    )(logits, dst, gmb)


def _scatter_body(ex_h, rdenp_h, src_h, dst_h, xlp_h, outp_h,
                  sidx, didx, rowbuf, rdrows, exbuf, albuf, msgbuf,
                  dumpbuf, sacc, sem1, sem2,
                  *, W, Wp, C, Wout, mean):
    cid = lax.axis_index("c")
    sid = lax.axis_index("s")
    wid = sid * 2 + cid
    ebase = wid * EW
    iota = lax.iota(jnp.int32, 16)
    nv = Wout // 16

    def zb(i, _):
        r = i // nv
        j = i - r * nv
        dumpbuf[r, pl.ds(j * 16, 16)] = jnp.zeros((16,), jnp.float32)
        return 0

    lax.fori_loop(0, 40 * nv, zb, 0)

    @pl.when(sid < 10)
    def _():
        for t in range(25):
            pltpu.sync_copy(dumpbuf, sacc.at[pl.ds(sid * 1000 + t * 40, 40)])

    plsc.subcore_barrier()

    scale = 1.0 / H if mean else 1.0

    def chunk(k, _):
        base = ebase + k * CH
        pltpu.sync_copy(src_h.at[pl.ds(base, CH)], sidx)
        pltpu.sync_copy(dst_h.at[pl.ds(base, CH)], didx)
        pltpu.sync_copy(ex_h.at[pl.ds(base * H, CH * H)], exbuf)
        cp1 = pltpu.async_copy(xlp_h.at[sidx], rowbuf, sem1)
        cp2 = pltpu.async_copy(rdenp_h.at[didx], rdrows, sem2)
        cp1.wait()
        cp2.wait()
        for gi in range(NG):
            e16 = gi * 16 + iota
            for h in range(H):
                rd = plsc.load_gather(rdrows, [e16, jnp.full((16,), h, jnp.int32)])
                ev = plsc.load_gather(exbuf, [e16 * H + h])
                a16 = ev * rd * scale
                plsc.store_scatter(albuf, [e16 * H + h], a16)

        def ebody(e, _):
            es = jnp.full((16,), 0, jnp.int32) + e * H
            avs = [plsc.load_gather(albuf, [es + h]) for h in range(H)]
            if not mean:
                for h in range(H):
                    for j in range(C // 16):
                        o = h * C + j * 16
                        msgbuf[e, pl.ds(o, 16)] = (rowbuf[e, pl.ds(o, 16)]
                                                   * avs[h])
            else:
                for j in range(C // 16):
                    acc = jnp.zeros((16,), jnp.float32)
                    for h in range(H):
                        acc = acc + rowbuf[e, pl.ds(h * C + j * 16, 16)] * avs[h]
                    msgbuf[e, pl.ds(j * 16, 16)] = acc
            return 0

        lax.fori_loop(0, CH, ebody, 0)
        pltpu.sync_copy(msgbuf, sacc.at[didx], add=True)
        return 0

    lax.fori_loop(0, NCH, chunk, 0)
    plsc.subcore_barrier()

    @pl.when(sid < 10)
    def _():
        for t in range(25):
            rbase = sid * 1000 + t * 40
            pltpu.sync_copy(sacc.at[pl.ds(rbase, 40)], dumpbuf)
            pltpu.sync_copy(dumpbuf, outp_h.at[cid, pl.ds(rbase, 40)])


def _sc_scatter(ex, rdenp, src, dst, xlp, mean):
    Wp = xlp.shape[1]
    W = Wp - 16
    C = W // H
    Wout = C if mean else W
    body = functools.partial(_scatter_body, W=W, Wp=Wp, C=C, Wout=Wout,
                             mean=mean)
    return pl.kernel(
        body,
        out_type=jax.ShapeDtypeStruct((2, N, Wout), jnp.float32),
        mesh=_mesh(),
        compiler_params=pltpu.CompilerParams(**_SC_PARAMS),
        scratch_types=[
            pltpu.VMEM((CH,), jnp.int32),
            pltpu.VMEM((CH,), jnp.int32),
            pltpu.VMEM((CH, Wp), jnp.float32),
            pltpu.VMEM((CH, 16), jnp.float32),
            pltpu.VMEM((CH * H,), jnp.float32),
            pltpu.VMEM((CH * H,), jnp.float32),
            pltpu.VMEM((CH, Wout), jnp.float32),
            pltpu.VMEM((40, Wout), jnp.float32),
            pltpu.VMEM_SHARED((N, Wout), jnp.float32),
            pltpu.SemaphoreType.DMA,
            pltpu.SemaphoreType.DMA,
        ],
    )(ex, rdenp, src, dst, xlp)


def _attbd(att):
    heads, c = att.shape
    m = heads * c
    idx = jnp.arange(m)
    return (jnp.zeros((m, heads), jnp.float32)
            .at[idx, idx // c].set(S1 * att.reshape(-1)))


def _gat_layer(xl, xr, al, ar, att, src, dst, mean):
    n, w = xl.shape
    pad = jnp.zeros((n, 12), jnp.float32)
    xlp = jnp.concatenate([xl, al, pad], axis=1)
    xrp = jnp.concatenate([xr, ar, pad], axis=1)
    atts2 = S2 * att.reshape(-1)
    logits, wmax = _sc_logits(xlp, xrp, src, dst, atts2)
    gm = jnp.max(wmax.reshape(NW, H, 16), axis=(0, 2))
    gmb = jnp.broadcast_to(gm[:, None], (H, 16)).reshape(H * 16)
    ex, denp = _sc_denom(logits, dst, gmb)
    rdenq = _rdenom(denp)
    rdenp = jnp.concatenate([rdenq.reshape(N, H),
                             jnp.zeros((N, 16 - H), jnp.float32)], axis=1)
    return _sc_scatter(ex, rdenp, src, dst, xlp, mean)


def kernel(x, edge_index, Wl1, bl1, Wr1, br1, att1, bias1, g1, b1,
           Wl2, bl2, Wr2, br2, att2, bias2, g2, b2):
    src = edge_index[0]
    dst = edge_index[1]

    xl1, xr1, al1, ar1 = _proj(x, Wl1, bl1, Wr1, br1, _attbd(att1))
    outp1 = _gat_layer(xl1, xr1, al1, ar1, att1, src, dst, mean=False)
    xl2, xr2, al2, ar2 = _combine_proj(outp1, bias1, g1, b1,
                                       Wl2, bl2, Wr2, br2, _attbd(att2))
    outp2 = _gat_layer(xl2, xr2, al2, ar2, att2, src, dst, mean=True)
    return _combine_epi(outp2, bias2, g2, b2)


# speedup vs baseline: 30.5328x; 1.5217x over previous
import functools

import jax
import jax.numpy as jnp
from jax import lax
from jax.experimental import pallas as pl
from jax.experimental.pallas import tpu as pltpu
from jax.experimental.pallas import tpu_sc as plsc

NEG_SLOPE_ATT = 0.2
NEG_SLOPE_ACT = 0.01
BN_EPS = 1e-5
N, E = 10000, 640000
H = 4
S1 = (1.0 + NEG_SLOPE_ATT) / 2.0
S2 = (1.0 - NEG_SLOPE_ATT) / 2.0

NW = 32
EW = E // NW
CH = 80
NCH = EW // CH
NG = CH // 16
DR = H * N // 16

_BN = 1000

_SC_PARAMS = dict(use_tc_tiling_on_sc=False, needs_layout_passes=False)


def _proj(x, Wl, bl, Wr, br, attbd):
    n, k = x.shape
    m = Wl.shape[1]

    def body(x_ref, wl_ref, bl_ref, wr_ref, br_ref, abd_ref,
             xl_ref, xr_ref, al_ref, ar_ref):
        xv = x_ref[...]
        xl = jnp.dot(xv, wl_ref[...], preferred_element_type=jnp.float32) + bl_ref[...]
        xr = jnp.dot(xv, wr_ref[...], preferred_element_type=jnp.float32) + br_ref[...]
        xl_ref[...] = xl
        xr_ref[...] = xr
        al_ref[...] = jnp.dot(xl, abd_ref[...], preferred_element_type=jnp.float32)
        ar_ref[...] = jnp.dot(xr, abd_ref[...], preferred_element_type=jnp.float32)

    return pl.pallas_call(
        body,
        grid=(n // _BN,),
        in_specs=[
            pl.BlockSpec((_BN, k), lambda i: (i, 0)),
            pl.BlockSpec((k, m), lambda i: (0, 0)),
            pl.BlockSpec((1, m), lambda i: (0, 0)),
            pl.BlockSpec((k, m), lambda i: (0, 0)),
            pl.BlockSpec((1, m), lambda i: (0, 0)),
            pl.BlockSpec((m, H), lambda i: (0, 0)),
        ],
        out_specs=[
            pl.BlockSpec((_BN, m), lambda i: (i, 0)),
            pl.BlockSpec((_BN, m), lambda i: (i, 0)),
            pl.BlockSpec((_BN, H), lambda i: (i, 0)),
            pl.BlockSpec((_BN, H), lambda i: (i, 0)),
        ],
        out_shape=[
            jax.ShapeDtypeStruct((n, m), jnp.float32),
            jax.ShapeDtypeStruct((n, m), jnp.float32),
            jax.ShapeDtypeStruct((n, H), jnp.float32),
            jax.ShapeDtypeStruct((n, H), jnp.float32),
        ],
    )(x, Wl, bl.reshape(1, m), Wr, br.reshape(1, m), attbd)


def _combine_proj(outp, bias, g, b, Wl, bl, Wr, br, attbd):
    _, n, w = outp.shape
    m = Wl.shape[1]

    def body(o_ref, bias_ref, g_ref, b_ref, wl_ref, bl_ref, wr_ref, br_ref,
             abd_ref, xl_ref, xr_ref, al_ref, ar_ref):
        h = o_ref[0] + o_ref[1] + bias_ref[...]
        h = jnp.where(h > 0, h, NEG_SLOPE_ACT * h)
        h = g_ref[...] * h / jnp.sqrt(1.0 + BN_EPS) + b_ref[...]
        xl = jnp.dot(h, wl_ref[...], preferred_element_type=jnp.float32) + bl_ref[...]
        xr = jnp.dot(h, wr_ref[...], preferred_element_type=jnp.float32) + br_ref[...]
        xl_ref[...] = xl
        xr_ref[...] = xr
        al_ref[...] = jnp.dot(xl, abd_ref[...], preferred_element_type=jnp.float32)
        ar_ref[...] = jnp.dot(xr, abd_ref[...], preferred_element_type=jnp.float32)

    return pl.pallas_call(
        body,
        grid=(n // _BN,),
        in_specs=[
            pl.BlockSpec((2, _BN, w), lambda i: (0, i, 0)),
            pl.BlockSpec((1, w), lambda i: (0, 0)),
            pl.BlockSpec((1, w), lambda i: (0, 0)),
            pl.BlockSpec((1, w), lambda i: (0, 0)),
            pl.BlockSpec((w, m), lambda i: (0, 0)),
            pl.BlockSpec((1, m), lambda i: (0, 0)),
            pl.BlockSpec((w, m), lambda i: (0, 0)),
            pl.BlockSpec((1, m), lambda i: (0, 0)),
            pl.BlockSpec((m, H), lambda i: (0, 0)),
        ],
        out_specs=[
            pl.BlockSpec((_BN, m), lambda i: (i, 0)),
            pl.BlockSpec((_BN, m), lambda i: (i, 0)),
            pl.BlockSpec((_BN, H), lambda i: (i, 0)),
            pl.BlockSpec((_BN, H), lambda i: (i, 0)),
        ],
        out_shape=[
            jax.ShapeDtypeStruct((n, m), jnp.float32),
            jax.ShapeDtypeStruct((n, m), jnp.float32),
            jax.ShapeDtypeStruct((n, H), jnp.float32),
            jax.ShapeDtypeStruct((n, H), jnp.float32),
        ],
    )(outp, bias.reshape(1, w), g.reshape(1, w), b.reshape(1, w),
      Wl, bl.reshape(1, m), Wr, br.reshape(1, m), attbd)


def _combine_epi(outp, bias, g, b):
    _, n, w = outp.shape

    def body(o_ref, bias_ref, g_ref, b_ref, out_ref):
        h = o_ref[0] + o_ref[1] + bias_ref[...]
        h = jnp.where(h > 0, h, NEG_SLOPE_ACT * h)
        out_ref[...] = g_ref[...] * h / jnp.sqrt(1.0 + BN_EPS) + b_ref[...]

    return pl.pallas_call(
        body,
        grid=(n // _BN,),
        in_specs=[
            pl.BlockSpec((2, _BN, w), lambda i: (0, i, 0)),
            pl.BlockSpec((1, w), lambda i: (0, 0)),
            pl.BlockSpec((1, w), lambda i: (0, 0)),
            pl.BlockSpec((1, w), lambda i: (0, 0)),
        ],
        out_specs=pl.BlockSpec((_BN, w), lambda i: (i, 0)),
        out_shape=jax.ShapeDtypeStruct((n, w), jnp.float32),
    )(outp, bias.reshape(1, w), g.reshape(1, w), b.reshape(1, w))


def _rdenom(denp):
    def body(d_ref, out_ref):
        out_ref[...] = 1.0 / (d_ref[0] + d_ref[1] + 1e-16)

    return pl.pallas_call(
        body,
        in_specs=[pl.BlockSpec((2, DR, 16), lambda: (0, 0, 0))],
        out_specs=pl.BlockSpec((DR, 16), lambda: (0, 0)),
        out_shape=jax.ShapeDtypeStruct((DR, 16), jnp.float32),
        grid=(),
    )(denp)


def _mesh():
    return plsc.VectorSubcoreMesh(core_axis_name="c", subcore_axis_name="s")


def _lo(w):
    return plsc.bitcast(lax.shift_left(w, 16), jnp.float32)


def _hi(w):
    mask = jnp.full((16,), -65536, jnp.int32)
    return plsc.bitcast(lax.bitwise_and(w, mask), jnp.float32)


def _logits_body(xlp, xrp, src_h, dst_h, atts2_h, log_h, wmax_h,
                 sidx, didx, xrow, yrow, logbuf, attbuf, wmaxbuf, sem1, sem2,
                 *, W, Wp, C):
    cid = lax.axis_index("c")
    sid = lax.axis_index("s")
    wid = sid * 2 + cid
    ebase = wid * EW
    pltpu.sync_copy(atts2_h, attbuf)
    iota = lax.iota(jnp.int32, 16)
    neg = jnp.full((16,), -3e38, jnp.float32)

    def chunk(k, mx):
        base = ebase + k * CH
        pltpu.sync_copy(src_h.at[pl.ds(base, CH)], sidx)
        pltpu.sync_copy(dst_h.at[pl.ds(base, CH)], didx)
        cp1 = pltpu.async_copy(xlp.at[sidx], xrow, sem1)
        cp2 = pltpu.async_copy(xrp.at[didx], yrow, sem2)
        cp1.wait()
        cp2.wait()
        mx = list(mx)
        e16s = [gi * 16 + iota for gi in range(NG)]
        for h in range(H):
            ch = jnp.full((16,), W // 2 + h // 2, jnp.int32)
            half = _lo if h % 2 == 0 else _hi
            accs = tuple(half(plsc.load_gather(xrow, [e16s[gi], ch]))
                         + half(plsc.load_gather(yrow, [e16s[gi], ch]))
                         for gi in range(NG))

            def cbody(c, accs):
                cs = jnp.full((16,), 0, jnp.int32) + c
                aw = plsc.load_gather(attbuf, [cs])
                alo = _lo(aw)
                ahi = _hi(aw)
                out = []
                for gi in range(NG):
                    xw = plsc.load_gather(xrow, [e16s[gi], cs])
                    yw = plsc.load_gather(yrow, [e16s[gi], cs])
                    zlo = _lo(xw) + _lo(yw)
                    zhi = _hi(xw) + _hi(yw)
                    out.append(accs[gi] + jnp.abs(zlo) * alo
                               + jnp.abs(zhi) * ahi)
                return tuple(out)

            accs = lax.fori_loop(h * (C // 2), (h + 1) * (C // 2), cbody,
                                 accs, unroll=4)
            for gi in range(NG):
                plsc.store_scatter(logbuf, [e16s[gi] * H + h], accs[gi])
                mx[h] = jnp.maximum(mx[h], accs[gi])
        pltpu.sync_copy(logbuf, log_h.at[pl.ds(base * H, CH * H)])
        return tuple(mx)

    mx = lax.fori_loop(0, NCH, chunk, (neg, neg, neg, neg))
    for h in range(H):
        wmaxbuf[pl.ds(h * 16, 16)] = mx[h]
    pltpu.sync_copy(wmaxbuf, wmax_h.at[pl.ds(wid * (H * 16), H * 16)])


def _sc_logits(xlp, xrp, src, dst, atts2):
    Wp2 = xlp.shape[1]
    W = Wp2 * 2 - 16
    C = W // H
    body = functools.partial(_logits_body, W=W, Wp=Wp2, C=C)
    return pl.kernel(
        body,
        out_type=[
            jax.ShapeDtypeStruct((E * H,), jnp.float32),
            jax.ShapeDtypeStruct((NW * H * 16,), jnp.float32),
        ],
        mesh=_mesh(),
        compiler_params=pltpu.CompilerParams(**_SC_PARAMS),
        scratch_types=[
            pltpu.VMEM((CH,), jnp.int32),
            pltpu.VMEM((CH,), jnp.int32),
            pltpu.VMEM((CH, Wp2), jnp.int32),
            pltpu.VMEM((CH, Wp2), jnp.int32),
            pltpu.VMEM((CH * H,), jnp.float32),
            pltpu.VMEM((W // 2,), jnp.int32),
            pltpu.VMEM((H * 16,), jnp.float32),
            pltpu.SemaphoreType.DMA,
            pltpu.SemaphoreType.DMA,
        ],
    )(xlp, xrp, src, dst, atts2)


def _denom_body(log_h, dst_h, gmb_h, ex_h, denp_h,
                didx, lbuf, exbuf, gbuf, denloc, idxbuf, sdden):
    cid = lax.axis_index("c")
    sid = lax.axis_index("s")
    wid = sid * 2 + cid
    ebase = wid * EW
    iota = lax.iota(jnp.int32, 16)
    pltpu.sync_copy(gmb_h, gbuf)

    def zr(i, _):
        denloc[i, :] = jnp.zeros((16,), jnp.float32)
        return 0

    lax.fori_loop(0, DR, zr, 0)

    for t in range(20):
        for jj in range(8):
            st = min(jj * 16, 125 - 16)
            idxbuf[t, pl.ds(st, 16)] = t * 125 + st + iota

    @pl.when(sid == 0)
    def _():
        pltpu.sync_copy(denloc, sdden)

    def chunk(k, _):
        base = ebase + k * CH
        pltpu.sync_copy(dst_h.at[pl.ds(base, CH)], didx)
        pltpu.sync_copy(log_h.at[pl.ds(base * H, CH * H)], lbuf)
        for gi in range(NG):
            e16 = gi * 16 + iota
            d16 = didx[pl.ds(gi * 16, 16)]
            for h in range(H):
                l16 = plsc.load_gather(lbuf, [e16 * H + h])
                ev = jnp.exp(l16 - gbuf[pl.ds(h * 16, 16)])
                plsc.store_scatter(exbuf, [e16 * H + h], ev)
                f16 = d16 * H + h
                plsc.addupdate_scatter(
                    denloc,
                    [lax.shift_right_logical(f16, 4),
                     lax.bitwise_and(f16, 15)], ev)
        pltpu.sync_copy(exbuf, ex_h.at[pl.ds(base * H, CH * H)])
        return 0

    lax.fori_loop(0, NCH, chunk, 0)
    plsc.subcore_barrier()
    for t in range(20):
        pltpu.sync_copy(denloc.at[pl.ds(t * 125, 125)],
                        sdden.at[idxbuf.at[t]], add=True)
    plsc.subcore_barrier()

    @pl.when(sid == 0)
    def _():
        pltpu.sync_copy(sdden, denloc)
        pltpu.sync_copy(denloc, denp_h.at[cid])


def _sc_denom(logits, dst, gmb):
    return pl.kernel(
        _denom_body,
        out_type=[
            jax.ShapeDtypeStruct((E * H,), jnp.float32),
            jax.ShapeDtypeStruct((2, DR, 16), jnp.float32),
        ],
        mesh=_mesh(),
        compiler_params=pltpu.CompilerParams(**_SC_PARAMS),
        scratch_types=[
            pltpu.VMEM((CH,), jnp.int32),
            pltpu.VMEM((CH * H,), jnp.float32),
            pltpu.VMEM((CH * H,), jnp.float32),
            pltpu.VMEM((H * 16,), jnp.float32),
            pltpu.VMEM((DR, 16), jnp.float32),
            pltpu.VMEM((20, 125), jnp.int32),
            pltpu.VMEM_SHARED((DR, 16), jnp.float32),
        ],
    )(logits, dst, gmb)


def _scatter_body(ex_h, rdenp_h, src_h, dst_h, xlp_h, outp_h,
                  sidx, didx, rowbuf, rdrows, exbuf, albuf, msgbuf,
                  dumpbuf, sacc, sem1, sem2,
                  *, W, Wp, C, Wout, mean):
    cid = lax.axis_index("c")
    sid = lax.axis_index("s")
    wid = sid * 2 + cid
    ebase = wid * EW
    iota = lax.iota(jnp.int32, 16)
    nv = Wout // 16

    def zb(i, _):
        r = i // nv
        j = i - r * nv
        dumpbuf[r, pl.ds(j * 16, 16)] = jnp.zeros((16,), jnp.float32)
        return 0

    lax.fori_loop(0, 40 * nv, zb, 0)

    @pl.when(sid < 10)
    def _():
        for t in range(25):
            pltpu.sync_copy(dumpbuf, sacc.at[pl.ds(sid * 1000 + t * 40, 40)])

    plsc.subcore_barrier()

    scale = 1.0 / H if mean else 1.0

    def chunk(k, _):
        base = ebase + k * CH
        pltpu.sync_copy(src_h.at[pl.ds(base, CH)], sidx)
        pltpu.sync_copy(dst_h.at[pl.ds(base, CH)], didx)
        pltpu.sync_copy(ex_h.at[pl.ds(base * H, CH * H)], exbuf)
        cp1 = pltpu.async_copy(xlp_h.at[sidx], rowbuf, sem1)
        cp2 = pltpu.async_copy(rdenp_h.at[didx], rdrows, sem2)
        cp1.wait()
        cp2.wait()
        for gi in range(NG):
            e16 = gi * 16 + iota
            for h in range(H):
                rd = plsc.load_gather(rdrows, [e16, jnp.full((16,), h, jnp.int32)])
                ev = plsc.load_gather(exbuf, [e16 * H + h])
                a16 = ev * rd * scale
                plsc.store_scatter(albuf, [e16 * H + h], a16)

        def ebody(e, _):
            es = jnp.full((16,), 0, jnp.int32) + e * H
            avs = [plsc.load_gather(albuf, [es + h]) for h in range(H)]
            if not mean:
                for h in range(H):
                    for jb in range(C // 32):
                        w0 = h * (C // 2) + jb * 16
                        cb = h * C + jb * 32
                        xw = rowbuf[e, pl.ds(w0, 16)]
                        msgbuf[e, pl.ds(cb, 16)] = _lo(xw) * avs[h]
                        msgbuf[e, pl.ds(cb + 16, 16)] = _hi(xw) * avs[h]
            else:
                for jb in range(C // 32):
                    acclo = jnp.zeros((16,), jnp.float32)
                    acchi = jnp.zeros((16,), jnp.float32)
                    for h in range(H):
                        xw = rowbuf[e, pl.ds(h * (C // 2) + jb * 16, 16)]
                        acclo = acclo + _lo(xw) * avs[h]
                        acchi = acchi + _hi(xw) * avs[h]
                    msgbuf[e, pl.ds(jb * 32, 16)] = acclo
                    msgbuf[e, pl.ds(jb * 32 + 16, 16)] = acchi
            return 0

        lax.fori_loop(0, CH, ebody, 0)
        pltpu.sync_copy(msgbuf, sacc.at[didx], add=True)
        return 0

    lax.fori_loop(0, NCH, chunk, 0)
    plsc.subcore_barrier()

    @pl.when(sid < 10)
    def _():
        for t in range(25):
            rbase = sid * 1000 + t * 40
            pltpu.sync_copy(sacc.at[pl.ds(rbase, 40)], dumpbuf)
            pltpu.sync_copy(dumpbuf, outp_h.at[cid, pl.ds(rbase, 40)])


def _sc_scatter(ex, rdenp, src, dst, xlp, mean):
    Wp2 = xlp.shape[1]
    W = Wp2 * 2 - 16
    C = W // H
    Wout = C if mean else W
    body = functools.partial(_scatter_body, W=W, Wp=Wp2, C=C, Wout=Wout,
                             mean=mean)
    return pl.kernel(
        body,
        out_type=jax.ShapeDtypeStruct((2, N, Wout), jnp.float32),
        mesh=_mesh(),
        compiler_params=pltpu.CompilerParams(**_SC_PARAMS),
        scratch_types=[
            pltpu.VMEM((CH,), jnp.int32),
            pltpu.VMEM((CH,), jnp.int32),
            pltpu.VMEM((CH, Wp2), jnp.int32),
            pltpu.VMEM((CH, 16), jnp.float32),
            pltpu.VMEM((CH * H,), jnp.float32),
            pltpu.VMEM((CH * H,), jnp.float32),
            pltpu.VMEM((CH, Wout), jnp.float32),
            pltpu.VMEM((40, Wout), jnp.float32),
            pltpu.VMEM_SHARED((N, Wout), jnp.float32),
            pltpu.SemaphoreType.DMA,
            pltpu.SemaphoreType.DMA,
        ],
    )(ex, rdenp, src, dst, xlp)


def _attbd(att):
    heads, c = att.shape
    m = heads * c
    idx = jnp.arange(m)
    return (jnp.zeros((m, heads), jnp.float32)
            .at[idx, idx // c].set(S1 * att.reshape(-1)))


def _pack16(x):
    n, w = x.shape
    return lax.bitcast_convert_type(
        x.astype(jnp.bfloat16).reshape(n, w // 2, 2), jnp.int32)


def _deperm(outp):
    b, n, w = outp.shape
    return outp.reshape(b, n, w // 32, 2, 16).swapaxes(-1, -2).reshape(b, n, w)


def _gat_layer(xl, xr, al, ar, att, src, dst, mean):
    n, w = xl.shape
    pad = jnp.zeros((n, 12), jnp.float32)
    xlp = _pack16(jnp.concatenate([xl, al, pad], axis=1))
    xrp = _pack16(jnp.concatenate([xr, ar, pad], axis=1))
    atts2 = _pack16((S2 * att.reshape(-1)).reshape(1, w)).reshape(w // 2)
    logits, wmax = _sc_logits(xlp, xrp, src, dst, atts2)
    gm = jnp.max(wmax.reshape(NW, H, 16), axis=(0, 2))
    gmb = jnp.broadcast_to(gm[:, None], (H, 16)).reshape(H * 16)
    ex, denp = _sc_denom(logits, dst, gmb)
    rdenq = _rdenom(denp)
    rdenp = jnp.concatenate([rdenq.reshape(N, H),
                             jnp.zeros((N, 16 - H), jnp.float32)], axis=1)
    return _deperm(_sc_scatter(ex, rdenp, src, dst, xlp, mean))


def kernel(x, edge_index, Wl1, bl1, Wr1, br1, att1, bias1, g1, b1,
           Wl2, bl2, Wr2, br2, att2, bias2, g2, b2):
    src = edge_index[0]
    dst = edge_index[1]

    xl1, xr1, al1, ar1 = _proj(x, Wl1, bl1, Wr1, br1, _attbd(att1))
    outp1 = _gat_layer(xl1, xr1, al1, ar1, att1, src, dst, mean=False)
    xl2, xr2, al2, ar2 = _combine_proj(outp1, bias1, g1, b1,
                                       Wl2, bl2, Wr2, br2, _attbd(att2))
    outp2 = _gat_layer(xl2, xr2, al2, ar2, att2, src, dst, mean=True)
    return _combine_epi(outp2, bias2, g2, b2)


# speedup vs baseline: 36.5691x; 1.1977x over previous
import functools

import jax
import jax.numpy as jnp
from jax import lax
from jax.experimental import pallas as pl
from jax.experimental.pallas import tpu as pltpu
from jax.experimental.pallas import tpu_sc as plsc

NEG_SLOPE_ATT = 0.2
NEG_SLOPE_ACT = 0.01
BN_EPS = 1e-5
N, E = 10000, 640000
H = 4
S1 = (1.0 + NEG_SLOPE_ATT) / 2.0
S2 = (1.0 - NEG_SLOPE_ATT) / 2.0

NW = 32
EW = E // NW
CH = 80
NCH = EW // CH
NG = CH // 16
DR = H * N // 16

_BN = 1000

_SC_PARAMS = dict(use_tc_tiling_on_sc=False, needs_layout_passes=False)


def _proj(x, Wl, bl, Wr, br, attbd):
    n, k = x.shape
    m = Wl.shape[1]

    def body(x_ref, wl_ref, bl_ref, wr_ref, br_ref, abd_ref,
             xl_ref, xr_ref, al_ref, ar_ref):
        xv = x_ref[...]
        xl = jnp.dot(xv, wl_ref[...], preferred_element_type=jnp.float32) + bl_ref[...]
        xr = jnp.dot(xv, wr_ref[...], preferred_element_type=jnp.float32) + br_ref[...]
        xl_ref[...] = xl
        xr_ref[...] = xr
        al_ref[...] = jnp.dot(xl, abd_ref[...], preferred_element_type=jnp.float32)
        ar_ref[...] = jnp.dot(xr, abd_ref[...], preferred_element_type=jnp.float32)

    return pl.pallas_call(
        body,
        grid=(n // _BN,),
        in_specs=[
            pl.BlockSpec((_BN, k), lambda i: (i, 0)),
            pl.BlockSpec((k, m), lambda i: (0, 0)),
            pl.BlockSpec((1, m), lambda i: (0, 0)),
            pl.BlockSpec((k, m), lambda i: (0, 0)),
            pl.BlockSpec((1, m), lambda i: (0, 0)),
            pl.BlockSpec((m, H), lambda i: (0, 0)),
        ],
        out_specs=[
            pl.BlockSpec((_BN, m), lambda i: (i, 0)),
            pl.BlockSpec((_BN, m), lambda i: (i, 0)),
            pl.BlockSpec((_BN, H), lambda i: (i, 0)),
            pl.BlockSpec((_BN, H), lambda i: (i, 0)),
        ],
        out_shape=[
            jax.ShapeDtypeStruct((n, m), jnp.float32),
            jax.ShapeDtypeStruct((n, m), jnp.float32),
            jax.ShapeDtypeStruct((n, H), jnp.float32),
            jax.ShapeDtypeStruct((n, H), jnp.float32),
        ],
    )(x, Wl, bl.reshape(1, m), Wr, br.reshape(1, m), attbd)


def _combine_proj(outp, bias, g, b, Wl, bl, Wr, br, attbd):
    _, n, w = outp.shape
    m = Wl.shape[1]

    def body(o_ref, bias_ref, g_ref, b_ref, wl_ref, bl_ref, wr_ref, br_ref,
             abd_ref, xl_ref, xr_ref, al_ref, ar_ref):
        h = o_ref[0] + o_ref[1] + bias_ref[...]
        h = jnp.where(h > 0, h, NEG_SLOPE_ACT * h)
        h = g_ref[...] * h / jnp.sqrt(1.0 + BN_EPS) + b_ref[...]
        xl = jnp.dot(h, wl_ref[...], preferred_element_type=jnp.float32) + bl_ref[...]
        xr = jnp.dot(h, wr_ref[...], preferred_element_type=jnp.float32) + br_ref[...]
        xl_ref[...] = xl
        xr_ref[...] = xr
        al_ref[...] = jnp.dot(xl, abd_ref[...], preferred_element_type=jnp.float32)
        ar_ref[...] = jnp.dot(xr, abd_ref[...], preferred_element_type=jnp.float32)

    return pl.pallas_call(
        body,
        grid=(n // _BN,),
        in_specs=[
            pl.BlockSpec((2, _BN, w), lambda i: (0, i, 0)),
            pl.BlockSpec((1, w), lambda i: (0, 0)),
            pl.BlockSpec((1, w), lambda i: (0, 0)),
            pl.BlockSpec((1, w), lambda i: (0, 0)),
            pl.BlockSpec((w, m), lambda i: (0, 0)),
            pl.BlockSpec((1, m), lambda i: (0, 0)),
            pl.BlockSpec((w, m), lambda i: (0, 0)),
            pl.BlockSpec((1, m), lambda i: (0, 0)),
            pl.BlockSpec((m, H), lambda i: (0, 0)),
        ],
        out_specs=[
            pl.BlockSpec((_BN, m), lambda i: (i, 0)),
            pl.BlockSpec((_BN, m), lambda i: (i, 0)),
            pl.BlockSpec((_BN, H), lambda i: (i, 0)),
            pl.BlockSpec((_BN, H), lambda i: (i, 0)),
        ],
        out_shape=[
            jax.ShapeDtypeStruct((n, m), jnp.float32),
            jax.ShapeDtypeStruct((n, m), jnp.float32),
            jax.ShapeDtypeStruct((n, H), jnp.float32),
            jax.ShapeDtypeStruct((n, H), jnp.float32),
        ],
    )(outp, bias.reshape(1, w), g.reshape(1, w), b.reshape(1, w),
      Wl, bl.reshape(1, m), Wr, br.reshape(1, m), attbd)


def _combine_epi(outp, bias, g, b):
    _, n, w = outp.shape

    def body(o_ref, bias_ref, g_ref, b_ref, out_ref):
        h = o_ref[0] + o_ref[1] + bias_ref[...]
        h = jnp.where(h > 0, h, NEG_SLOPE_ACT * h)
        out_ref[...] = g_ref[...] * h / jnp.sqrt(1.0 + BN_EPS) + b_ref[...]

    return pl.pallas_call(
        body,
        grid=(n // _BN,),
        in_specs=[
            pl.BlockSpec((2, _BN, w), lambda i: (0, i, 0)),
            pl.BlockSpec((1, w), lambda i: (0, 0)),
            pl.BlockSpec((1, w), lambda i: (0, 0)),
            pl.BlockSpec((1, w), lambda i: (0, 0)),
        ],
        out_specs=pl.BlockSpec((_BN, w), lambda i: (i, 0)),
        out_shape=jax.ShapeDtypeStruct((n, w), jnp.float32),
    )(outp, bias.reshape(1, w), g.reshape(1, w), b.reshape(1, w))


def _rdenom(denp):
    def body(d_ref, out_ref):
        out_ref[...] = 1.0 / (d_ref[0] + d_ref[1] + 1e-16)

    return pl.pallas_call(
        body,
        in_specs=[pl.BlockSpec((2, DR, 16), lambda: (0, 0, 0))],
        out_specs=pl.BlockSpec((DR, 16), lambda: (0, 0)),
        out_shape=jax.ShapeDtypeStruct((DR, 16), jnp.float32),
        grid=(),
    )(denp)


def _mesh():
    return plsc.VectorSubcoreMesh(core_axis_name="c", subcore_axis_name="s")


def _lo(w):
    return plsc.bitcast(lax.shift_left(w, 16), jnp.float32)


def _hi(w):
    mask = jnp.full((16,), -65536, jnp.int32)
    return plsc.bitcast(lax.bitwise_and(w, mask), jnp.float32)


def _logits_body(xlp, xrp, src_h, dst_h, atts2_h, log_h, wmax_h,
                 sidx, didx, xrow, yrow, sidx2, didx2, xrow2, yrow2,
                 logbuf, attbuf, wmaxbuf, sem1, sem2,
                 *, W, Wp, C):
    cid = lax.axis_index("c")
    sid = lax.axis_index("s")
    wid = sid * 2 + cid
    ebase = wid * EW
    pltpu.sync_copy(atts2_h, attbuf)
    iota = lax.iota(jnp.int32, 16)
    neg = jnp.full((16,), -3e38, jnp.float32)

    def issue(k, sidx, didx, xrow, yrow, sem):
        base = ebase + k * CH
        pltpu.sync_copy(src_h.at[pl.ds(base, CH)], sidx)
        pltpu.sync_copy(dst_h.at[pl.ds(base, CH)], didx)
        pltpu.async_copy(xlp.at[sidx], xrow, sem)
        pltpu.async_copy(xrp.at[didx], yrow, sem)

    def wait(sidx, didx, xrow, yrow, sem):
        pltpu.make_async_copy(xlp.at[sidx], xrow, sem).wait()
        pltpu.make_async_copy(xrp.at[didx], yrow, sem).wait()

    def compute(k, xrow, yrow, mx):
        base = ebase + k * CH
        mx = list(mx)
        e16s = [gi * 16 + iota for gi in range(NG)]
        for h in range(H):
            ch = jnp.full((16,), W // 2 + h // 2, jnp.int32)
            half = _lo if h % 2 == 0 else _hi
            accs = tuple(half(plsc.load_gather(xrow, [e16s[gi], ch]))
                         + half(plsc.load_gather(yrow, [e16s[gi], ch]))
                         for gi in range(NG))

            def cbody(c, accs):
                cs = jnp.full((16,), 0, jnp.int32) + c
                aw = plsc.load_gather(attbuf, [cs])
                alo = _lo(aw)
                ahi = _hi(aw)
                out = []
                for gi in range(NG):
                    xw = plsc.load_gather(xrow, [e16s[gi], cs])
                    yw = plsc.load_gather(yrow, [e16s[gi], cs])
                    zlo = _lo(xw) + _lo(yw)
                    zhi = _hi(xw) + _hi(yw)
                    out.append(accs[gi] + jnp.abs(zlo) * alo
                               + jnp.abs(zhi) * ahi)
                return tuple(out)

            accs = lax.fori_loop(h * (C // 2), (h + 1) * (C // 2), cbody,
                                 accs, unroll=4)
            for gi in range(NG):
                plsc.store_scatter(logbuf, [e16s[gi] * H + h], accs[gi])
                mx[h] = jnp.maximum(mx[h], accs[gi])
        pltpu.sync_copy(logbuf, log_h.at[pl.ds(base * H, CH * H)])
        return tuple(mx)

    bufA = (sidx, didx, xrow, yrow, sem1)
    bufB = (sidx2, didx2, xrow2, yrow2, sem2)
    issue(0, *bufA)

    def body(m, mx):
        issue(2 * m + 1, *bufB)
        wait(*bufA)
        mx = compute(2 * m, bufA[2], bufA[3], mx)

        @pl.when(2 * m + 2 < NCH)
        def _():
            issue(2 * m + 2, *bufA)

        wait(*bufB)
        return compute(2 * m + 1, bufB[2], bufB[3], mx)

    mx = lax.fori_loop(0, NCH // 2, body, (neg, neg, neg, neg))
    for h in range(H):
        wmaxbuf[pl.ds(h * 16, 16)] = mx[h]
    pltpu.sync_copy(wmaxbuf, wmax_h.at[pl.ds(wid * (H * 16), H * 16)])


def _sc_logits(xlp, xrp, src, dst, atts2):
    Wp2 = xlp.shape[1]
    W = Wp2 * 2 - 16
    C = W // H
    body = functools.partial(_logits_body, W=W, Wp=Wp2, C=C)
    return pl.kernel(
        body,
        out_type=[
            jax.ShapeDtypeStruct((E * H,), jnp.float32),
            jax.ShapeDtypeStruct((NW * H * 16,), jnp.float32),
        ],
        mesh=_mesh(),
        compiler_params=pltpu.CompilerParams(**_SC_PARAMS),
        scratch_types=[
            pltpu.VMEM((CH,), jnp.int32),
            pltpu.VMEM((CH,), jnp.int32),
            pltpu.VMEM((CH, Wp2), jnp.int32),
            pltpu.VMEM((CH, Wp2), jnp.int32),
            pltpu.VMEM((CH,), jnp.int32),
            pltpu.VMEM((CH,), jnp.int32),
            pltpu.VMEM((CH, Wp2), jnp.int32),
            pltpu.VMEM((CH, Wp2), jnp.int32),
            pltpu.VMEM((CH * H,), jnp.float32),
            pltpu.VMEM((W // 2,), jnp.int32),
            pltpu.VMEM((H * 16,), jnp.float32),
            pltpu.SemaphoreType.DMA,
            pltpu.SemaphoreType.DMA,
        ],
    )(xlp, xrp, src, dst, atts2)


def _denom_body(log_h, dst_h, gmb_h, ex_h, denp_h,
                didx, lbuf, exbuf, gbuf, denloc, idxbuf, sdden):
    cid = lax.axis_index("c")
    sid = lax.axis_index("s")
    wid = sid * 2 + cid
    ebase = wid * EW
    iota = lax.iota(jnp.int32, 16)
    pltpu.sync_copy(gmb_h, gbuf)

    def zr(i, _):
        denloc[i, :] = jnp.zeros((16,), jnp.float32)
        return 0

    lax.fori_loop(0, DR, zr, 0)

    for t in range(20):
        for jj in range(8):
            st = min(jj * 16, 125 - 16)
            idxbuf[t, pl.ds(st, 16)] = t * 125 + st + iota

    @pl.when(sid == 0)
    def _():
        pltpu.sync_copy(denloc, sdden)

    def chunk(k, _):
        base = ebase + k * CH
        pltpu.sync_copy(dst_h.at[pl.ds(base, CH)], didx)
        pltpu.sync_copy(log_h.at[pl.ds(base * H, CH * H)], lbuf)
        for gi in range(NG):
            e16 = gi * 16 + iota
            d16 = didx[pl.ds(gi * 16, 16)]
            for h in range(H):
                l16 = plsc.load_gather(lbuf, [e16 * H + h])
                ev = jnp.exp(l16 - gbuf[pl.ds(h * 16, 16)])
                plsc.store_scatter(exbuf, [e16 * H + h], ev)
                f16 = d16 * H + h
                plsc.addupdate_scatter(
                    denloc,
                    [lax.shift_right_logical(f16, 4),
                     lax.bitwise_and(f16, 15)], ev)
        pltpu.sync_copy(exbuf, ex_h.at[pl.ds(base * H, CH * H)])
        return 0

    lax.fori_loop(0, NCH, chunk, 0)
    plsc.subcore_barrier()
    for t in range(20):
        pltpu.sync_copy(denloc.at[pl.ds(t * 125, 125)],
                        sdden.at[idxbuf.at[t]], add=True)
    plsc.subcore_barrier()

    @pl.when(sid == 0)
    def _():
        pltpu.sync_copy(sdden, denloc)
        pltpu.sync_copy(denloc, denp_h.at[cid])


def _sc_denom(logits, dst, gmb):
    return pl.kernel(
        _denom_body,
        out_type=[
            jax.ShapeDtypeStruct((E * H,), jnp.float32),
            jax.ShapeDtypeStruct((2, DR, 16), jnp.float32),
        ],
        mesh=_mesh(),
        compiler_params=pltpu.CompilerParams(**_SC_PARAMS),
        scratch_types=[
            pltpu.VMEM((CH,), jnp.int32),
            pltpu.VMEM((CH * H,), jnp.float32),
            pltpu.VMEM((CH * H,), jnp.float32),
            pltpu.VMEM((H * 16,), jnp.float32),
            pltpu.VMEM((DR, 16), jnp.float32),
            pltpu.VMEM((20, 125), jnp.int32),
            pltpu.VMEM_SHARED((DR, 16), jnp.float32),
        ],
    )(logits, dst, gmb)


def _scatter_body(ex_h, rdenp_h, src_h, dst_h, xlp_h, outp_h,
                  sidx, didx, rowbuf, rdrows, exbuf,
                  sidx2, didx2, rowbuf2, rdrows2, exbuf2, albuf, msgbuf,
                  dumpbuf, sacc, sem1, sem2,
                  *, W, Wp, C, Wout, mean):
    cid = lax.axis_index("c")
    sid = lax.axis_index("s")
    wid = sid * 2 + cid
    ebase = wid * EW
    iota = lax.iota(jnp.int32, 16)
    nv = Wout // 16

    def zb(i, _):
        r = i // nv
        j = i - r * nv
        dumpbuf[r, pl.ds(j * 16, 16)] = jnp.zeros((16,), jnp.float32)
        return 0

    lax.fori_loop(0, 40 * nv, zb, 0)

    @pl.when(sid < 10)
    def _():
        for t in range(25):
            pltpu.sync_copy(dumpbuf, sacc.at[pl.ds(sid * 1000 + t * 40, 40)])

    plsc.subcore_barrier()

    scale = 1.0 / H if mean else 1.0

    def issue(k, sidx, didx, rowbuf, rdrows, exbuf, sem):
        base = ebase + k * CH
        pltpu.sync_copy(src_h.at[pl.ds(base, CH)], sidx)
        pltpu.sync_copy(dst_h.at[pl.ds(base, CH)], didx)
        pltpu.sync_copy(ex_h.at[pl.ds(base * H, CH * H)], exbuf)
        pltpu.async_copy(xlp_h.at[sidx], rowbuf, sem)
        pltpu.async_copy(rdenp_h.at[didx], rdrows, sem)

    def wait(sidx, didx, rowbuf, rdrows, exbuf, sem):
        pltpu.make_async_copy(xlp_h.at[sidx], rowbuf, sem).wait()
        pltpu.make_async_copy(rdenp_h.at[didx], rdrows, sem).wait()

    def compute(k, didx, rowbuf, rdrows, exbuf):
        for gi in range(NG):
            e16 = gi * 16 + iota
            for h in range(H):
                rd = plsc.load_gather(rdrows, [e16, jnp.full((16,), h, jnp.int32)])
                ev = plsc.load_gather(exbuf, [e16 * H + h])
                a16 = ev * rd * scale
                plsc.store_scatter(albuf, [e16 * H + h], a16)

        def ebody(e, _):
            es = jnp.full((16,), 0, jnp.int32) + e * H
            avs = [plsc.load_gather(albuf, [es + h]) for h in range(H)]
            if not mean:
                for h in range(H):
                    for jb in range(C // 32):
                        w0 = h * (C // 2) + jb * 16
                        cb = h * C + jb * 32
                        xw = rowbuf[e, pl.ds(w0, 16)]
                        msgbuf[e, pl.ds(cb, 16)] = _lo(xw) * avs[h]
                        msgbuf[e, pl.ds(cb + 16, 16)] = _hi(xw) * avs[h]
            else:
                for jb in range(C // 32):
                    acclo = jnp.zeros((16,), jnp.float32)
                    acchi = jnp.zeros((16,), jnp.float32)
                    for h in range(H):
                        xw = rowbuf[e, pl.ds(h * (C // 2) + jb * 16, 16)]
                        acclo = acclo + _lo(xw) * avs[h]
                        acchi = acchi + _hi(xw) * avs[h]
                    msgbuf[e, pl.ds(jb * 32, 16)] = acclo
                    msgbuf[e, pl.ds(jb * 32 + 16, 16)] = acchi
            return 0

        lax.fori_loop(0, CH, ebody, 0)
        pltpu.sync_copy(msgbuf, sacc.at[didx], add=True)

    bufA = (sidx, didx, rowbuf, rdrows, exbuf, sem1)
    bufB = (sidx2, didx2, rowbuf2, rdrows2, exbuf2, sem2)
    issue(0, *bufA)

    def body(m, _):
        issue(2 * m + 1, *bufB)
        wait(*bufA)
        compute(2 * m, bufA[1], bufA[2], bufA[3], bufA[4])

        @pl.when(2 * m + 2 < NCH)
        def _():
            issue(2 * m + 2, *bufA)

        wait(*bufB)
        compute(2 * m + 1, bufB[1], bufB[2], bufB[3], bufB[4])
        return 0

    lax.fori_loop(0, NCH // 2, body, 0)
    plsc.subcore_barrier()

    @pl.when(sid < 10)
    def _():
        for t in range(25):
            rbase = sid * 1000 + t * 40
            pltpu.sync_copy(sacc.at[pl.ds(rbase, 40)], dumpbuf)
            pltpu.sync_copy(dumpbuf, outp_h.at[cid, pl.ds(rbase, 40)])


def _sc_scatter(ex, rdenp, src, dst, xlp, mean):
    Wp2 = xlp.shape[1]
    W = Wp2 * 2 - 16
    C = W // H
    Wout = C if mean else W
    body = functools.partial(_scatter_body, W=W, Wp=Wp2, C=C, Wout=Wout,
                             mean=mean)
    return pl.kernel(
        body,
        out_type=jax.ShapeDtypeStruct((2, N, Wout), jnp.float32),
        mesh=_mesh(),
        compiler_params=pltpu.CompilerParams(**_SC_PARAMS),
        scratch_types=[
            pltpu.VMEM((CH,), jnp.int32),
            pltpu.VMEM((CH,), jnp.int32),
            pltpu.VMEM((CH, Wp2), jnp.int32),
            pltpu.VMEM((CH, 16), jnp.float32),
            pltpu.VMEM((CH * H,), jnp.float32),
            pltpu.VMEM((CH,), jnp.int32),
            pltpu.VMEM((CH,), jnp.int32),
            pltpu.VMEM((CH, Wp2), jnp.int32),
            pltpu.VMEM((CH, 16), jnp.float32),
            pltpu.VMEM((CH * H,), jnp.float32),
            pltpu.VMEM((CH * H,), jnp.float32),
            pltpu.VMEM((CH, Wout), jnp.float32),
            pltpu.VMEM((40, Wout), jnp.float32),
            pltpu.VMEM_SHARED((N, Wout), jnp.float32),
            pltpu.SemaphoreType.DMA,
            pltpu.SemaphoreType.DMA,
        ],
    )(ex, rdenp, src, dst, xlp)


def _attbd(att):
    heads, c = att.shape
    m = heads * c
    idx = jnp.arange(m)
    return (jnp.zeros((m, heads), jnp.float32)
            .at[idx, idx // c].set(S1 * att.reshape(-1)))


def _pack16(x):
    n, w = x.shape
    return lax.bitcast_convert_type(
        x.astype(jnp.bfloat16).reshape(n, w // 2, 2), jnp.int32)


def _deperm(outp):
    b, n, w = outp.shape
    return outp.reshape(b, n, w // 32, 2, 16).swapaxes(-1, -2).reshape(b, n, w)


def _gat_layer(xl, xr, al, ar, att, src, dst, mean):
    n, w = xl.shape
    pad = jnp.zeros((n, 12), jnp.float32)
    xlp = _pack16(jnp.concatenate([xl, al, pad], axis=1))
    xrp = _pack16(jnp.concatenate([xr, ar, pad], axis=1))
    atts2 = _pack16((S2 * att.reshape(-1)).reshape(1, w)).reshape(w // 2)
    logits, wmax = _sc_logits(xlp, xrp, src, dst, atts2)
    gm = jnp.max(wmax.reshape(NW, H, 16), axis=(0, 2))
    gmb = jnp.broadcast_to(gm[:, None], (H, 16)).reshape(H * 16)
    ex, denp = _sc_denom(logits, dst, gmb)
    rdenq = _rdenom(denp)
    rdenp = jnp.concatenate([rdenq.reshape(N, H),
                             jnp.zeros((N, 16 - H), jnp.float32)], axis=1)
    return _deperm(_sc_scatter(ex, rdenp, src, dst, xlp, mean))


def kernel(x, edge_index, Wl1, bl1, Wr1, br1, att1, bias1, g1, b1,
           Wl2, bl2, Wr2, br2, att2, bias2, g2, b2):
    src = edge_index[0]
    dst = edge_index[1]

    xl1, xr1, al1, ar1 = _proj(x, Wl1, bl1, Wr1, br1, _attbd(att1))
    outp1 = _gat_layer(xl1, xr1, al1, ar1, att1, src, dst, mean=False)
    xl2, xr2, al2, ar2 = _combine_proj(outp1, bias1, g1, b1,
                                       Wl2, bl2, Wr2, br2, _attbd(att2))
    outp2 = _gat_layer(xl2, xr2, al2, ar2, att2, src, dst, mean=True)
    return _combine_epi(outp2, bias2, g2, b2)


# speedup vs baseline: 39.8478x; 1.0897x over previous
import functools

import jax
import jax.numpy as jnp
from jax import lax
from jax.experimental import pallas as pl
from jax.experimental.pallas import tpu as pltpu
from jax.experimental.pallas import tpu_sc as plsc

NEG_SLOPE_ATT = 0.2
NEG_SLOPE_ACT = 0.01
BN_EPS = 1e-5
N, E = 10000, 640000
H = 4
S1 = (1.0 + NEG_SLOPE_ATT) / 2.0
S2 = (1.0 - NEG_SLOPE_ATT) / 2.0

NW = 32
EW = E // NW
CH = 80
NCH = EW // CH
NG = CH // 16
DR = H * N // 16

_BN = 1000

_SC_PARAMS = dict(use_tc_tiling_on_sc=False, needs_layout_passes=False)


def _proj(x, Wl, bl, Wr, br, attbd):
    n, k = x.shape
    m = Wl.shape[1]

    def body(x_ref, wl_ref, bl_ref, wr_ref, br_ref, abd_ref,
             xl_ref, xr_ref, al_ref, ar_ref):
        xv = x_ref[...]
        xl = jnp.dot(xv, wl_ref[...], preferred_element_type=jnp.float32) + bl_ref[...]
        xr = jnp.dot(xv, wr_ref[...], preferred_element_type=jnp.float32) + br_ref[...]
        xl_ref[...] = xl
        xr_ref[...] = xr
        al_ref[...] = jnp.dot(xl, abd_ref[...], preferred_element_type=jnp.float32)
        ar_ref[...] = jnp.dot(xr, abd_ref[...], preferred_element_type=jnp.float32)

    return pl.pallas_call(
        body,
        grid=(n // _BN,),
        in_specs=[
            pl.BlockSpec((_BN, k), lambda i: (i, 0)),
            pl.BlockSpec((k, m), lambda i: (0, 0)),
            pl.BlockSpec((1, m), lambda i: (0, 0)),
            pl.BlockSpec((k, m), lambda i: (0, 0)),
            pl.BlockSpec((1, m), lambda i: (0, 0)),
            pl.BlockSpec((m, H), lambda i: (0, 0)),
        ],
        out_specs=[
            pl.BlockSpec((_BN, m), lambda i: (i, 0)),
            pl.BlockSpec((_BN, m), lambda i: (i, 0)),
            pl.BlockSpec((_BN, H), lambda i: (i, 0)),
            pl.BlockSpec((_BN, H), lambda i: (i, 0)),
        ],
        out_shape=[
            jax.ShapeDtypeStruct((n, m), jnp.float32),
            jax.ShapeDtypeStruct((n, m), jnp.float32),
            jax.ShapeDtypeStruct((n, H), jnp.float32),
            jax.ShapeDtypeStruct((n, H), jnp.float32),
        ],
    )(x, Wl, bl.reshape(1, m), Wr, br.reshape(1, m), attbd)


def _combine_proj(outp, bias, g, b, Wl, bl, Wr, br, attbd):
    _, n, w = outp.shape
    m = Wl.shape[1]

    def body(o_ref, bias_ref, g_ref, b_ref, wl_ref, bl_ref, wr_ref, br_ref,
             abd_ref, xl_ref, xr_ref, al_ref, ar_ref):
        h = o_ref[0] + o_ref[1] + bias_ref[...]
        h = jnp.where(h > 0, h, NEG_SLOPE_ACT * h)
        h = g_ref[...] * h / jnp.sqrt(1.0 + BN_EPS) + b_ref[...]
        xl = jnp.dot(h, wl_ref[...], preferred_element_type=jnp.float32) + bl_ref[...]
        xr = jnp.dot(h, wr_ref[...], preferred_element_type=jnp.float32) + br_ref[...]
        xl_ref[...] = xl
        xr_ref[...] = xr
        al_ref[...] = jnp.dot(xl, abd_ref[...], preferred_element_type=jnp.float32)
        ar_ref[...] = jnp.dot(xr, abd_ref[...], preferred_element_type=jnp.float32)

    return pl.pallas_call(
        body,
        grid=(n // _BN,),
        in_specs=[
            pl.BlockSpec((2, _BN, w), lambda i: (0, i, 0)),
            pl.BlockSpec((1, w), lambda i: (0, 0)),
            pl.BlockSpec((1, w), lambda i: (0, 0)),
            pl.BlockSpec((1, w), lambda i: (0, 0)),
            pl.BlockSpec((w, m), lambda i: (0, 0)),
            pl.BlockSpec((1, m), lambda i: (0, 0)),
            pl.BlockSpec((w, m), lambda i: (0, 0)),
            pl.BlockSpec((1, m), lambda i: (0, 0)),
            pl.BlockSpec((m, H), lambda i: (0, 0)),
        ],
        out_specs=[
            pl.BlockSpec((_BN, m), lambda i: (i, 0)),
            pl.BlockSpec((_BN, m), lambda i: (i, 0)),
            pl.BlockSpec((_BN, H), lambda i: (i, 0)),
            pl.BlockSpec((_BN, H), lambda i: (i, 0)),
        ],
        out_shape=[
            jax.ShapeDtypeStruct((n, m), jnp.float32),
            jax.ShapeDtypeStruct((n, m), jnp.float32),
            jax.ShapeDtypeStruct((n, H), jnp.float32),
            jax.ShapeDtypeStruct((n, H), jnp.float32),
        ],
    )(outp, bias.reshape(1, w), g.reshape(1, w), b.reshape(1, w),
      Wl, bl.reshape(1, m), Wr, br.reshape(1, m), attbd)


def _combine_epi(outp, bias, g, b):
    _, n, w = outp.shape

    def body(o_ref, bias_ref, g_ref, b_ref, out_ref):
        h = o_ref[0] + o_ref[1] + bias_ref[...]
        h = jnp.where(h > 0, h, NEG_SLOPE_ACT * h)
        out_ref[...] = g_ref[...] * h / jnp.sqrt(1.0 + BN_EPS) + b_ref[...]

    return pl.pallas_call(
        body,
        grid=(n // _BN,),
        in_specs=[
            pl.BlockSpec((2, _BN, w), lambda i: (0, i, 0)),
            pl.BlockSpec((1, w), lambda i: (0, 0)),
            pl.BlockSpec((1, w), lambda i: (0, 0)),
            pl.BlockSpec((1, w), lambda i: (0, 0)),
        ],
        out_specs=pl.BlockSpec((_BN, w), lambda i: (i, 0)),
        out_shape=jax.ShapeDtypeStruct((n, w), jnp.float32),
    )(outp, bias.reshape(1, w), g.reshape(1, w), b.reshape(1, w))


def _rdenom(denp):
    def body(d_ref, out_ref):
        out_ref[...] = 1.0 / (d_ref[0] + d_ref[1] + 1e-16)

    return pl.pallas_call(
        body,
        in_specs=[pl.BlockSpec((2, DR, 16), lambda: (0, 0, 0))],
        out_specs=pl.BlockSpec((DR, 16), lambda: (0, 0)),
        out_shape=jax.ShapeDtypeStruct((DR, 16), jnp.float32),
        grid=(),
    )(denp)


def _mesh():
    return plsc.VectorSubcoreMesh(core_axis_name="c", subcore_axis_name="s")


def _lo(w):
    return plsc.bitcast(lax.shift_left(w, 16), jnp.float32)


def _hi(w):
    mask = jnp.full((16,), -65536, jnp.int32)
    return plsc.bitcast(lax.bitwise_and(w, mask), jnp.float32)


def _logits_body(xlp, xrp, src_h, dst_h, atts2_h, log_h, wmax_h,
                 sidx, didx, xrow, yrow, sidx2, didx2, xrow2, yrow2,
                 logbuf, attbuf, wmaxbuf, sem1, sem2,
                 *, W, Wp, C):
    cid = lax.axis_index("c")
    sid = lax.axis_index("s")
    wid = sid * 2 + cid
    ebase = wid * EW
    pltpu.sync_copy(atts2_h, attbuf)
    iota = lax.iota(jnp.int32, 16)
    neg = jnp.full((16,), -3e38, jnp.float32)

    def issue(k, sidx, didx, xrow, yrow, sem):
        base = ebase + k * CH
        pltpu.sync_copy(src_h.at[pl.ds(base, CH)], sidx)
        pltpu.sync_copy(dst_h.at[pl.ds(base, CH)], didx)
        pltpu.async_copy(xlp.at[sidx], xrow, sem)
        pltpu.async_copy(xrp.at[didx], yrow, sem)

    def wait(sidx, didx, xrow, yrow, sem):
        pltpu.make_async_copy(xlp.at[sidx], xrow, sem).wait()
        pltpu.make_async_copy(xrp.at[didx], yrow, sem).wait()

    def compute(k, xrow, yrow, mx):
        base = ebase + k * CH
        mx = list(mx)
        e16s = [gi * 16 + iota for gi in range(NG)]
        for h in range(H):
            ch = jnp.full((16,), W // 2 + h // 2, jnp.int32)
            half = _lo if h % 2 == 0 else _hi
            accs = tuple(half(plsc.load_gather(xrow, [e16s[gi], ch]))
                         + half(plsc.load_gather(yrow, [e16s[gi], ch]))
                         for gi in range(NG))

            def cbody(c, accs):
                cs = jnp.full((16,), 0, jnp.int32) + c
                aw = plsc.load_gather(attbuf, [cs])
                alo = _lo(aw)
                ahi = _hi(aw)
                out = []
                for gi in range(NG):
                    xw = plsc.load_gather(xrow, [e16s[gi], cs])
                    yw = plsc.load_gather(yrow, [e16s[gi], cs])
                    zlo = _lo(xw) + _lo(yw)
                    zhi = _hi(xw) + _hi(yw)
                    out.append(accs[gi] + jnp.abs(zlo) * alo
                               + jnp.abs(zhi) * ahi)
                return tuple(out)

            accs = lax.fori_loop(h * (C // 2), (h + 1) * (C // 2), cbody,
                                 accs, unroll=4)
            for gi in range(NG):
                plsc.store_scatter(logbuf, [e16s[gi] * H + h], accs[gi])
                mx[h] = jnp.maximum(mx[h], accs[gi])
        pltpu.sync_copy(logbuf, log_h.at[pl.ds(base * H, CH * H)])
        return tuple(mx)

    bufA = (sidx, didx, xrow, yrow, sem1)
    bufB = (sidx2, didx2, xrow2, yrow2, sem2)
    issue(0, *bufA)

    def body(m, mx):
        issue(2 * m + 1, *bufB)
        wait(*bufA)
        mx = compute(2 * m, bufA[2], bufA[3], mx)

        @pl.when(2 * m + 2 < NCH)
        def _():
            issue(2 * m + 2, *bufA)

        wait(*bufB)
        return compute(2 * m + 1, bufB[2], bufB[3], mx)

    mx = lax.fori_loop(0, NCH // 2, body, (neg, neg, neg, neg))
    for h in range(H):
        wmaxbuf[pl.ds(h * 16, 16)] = mx[h]
    pltpu.sync_copy(wmaxbuf, wmax_h.at[pl.ds(wid * (H * 16), H * 16)])


def _sc_logits(xlp, xrp, src, dst, atts2):
    Wp2 = xlp.shape[1]
    W = Wp2 * 2 - 16
    C = W // H
    body = functools.partial(_logits_body, W=W, Wp=Wp2, C=C)
    return pl.kernel(
        body,
        out_type=[
            jax.ShapeDtypeStruct((E * H,), jnp.float32),
            jax.ShapeDtypeStruct((NW * H * 16,), jnp.float32),
        ],
        mesh=_mesh(),
        compiler_params=pltpu.CompilerParams(**_SC_PARAMS),
        scratch_types=[
            pltpu.VMEM((CH,), jnp.int32),
            pltpu.VMEM((CH,), jnp.int32),
            pltpu.VMEM((CH, Wp2), jnp.int32),
            pltpu.VMEM((CH, Wp2), jnp.int32),
            pltpu.VMEM((CH,), jnp.int32),
            pltpu.VMEM((CH,), jnp.int32),
            pltpu.VMEM((CH, Wp2), jnp.int32),
            pltpu.VMEM((CH, Wp2), jnp.int32),
            pltpu.VMEM((CH * H,), jnp.float32),
            pltpu.VMEM((W // 2,), jnp.int32),
            pltpu.VMEM((H * 16,), jnp.float32),
            pltpu.SemaphoreType.DMA,
            pltpu.SemaphoreType.DMA,
        ],
    )(xlp, xrp, src, dst, atts2)


def _denom_body(log_h, dst_h, gmb_h, ex_h, denp_h,
                didx, lbuf, didx2, lbuf2, exbuf, gbuf, denloc, idxbuf,
                sdden, sem1, sem2):
    cid = lax.axis_index("c")
    sid = lax.axis_index("s")
    wid = sid * 2 + cid
    ebase = wid * EW
    iota = lax.iota(jnp.int32, 16)
    pltpu.sync_copy(gmb_h, gbuf)

    def zr(i, _):
        denloc[i, :] = jnp.zeros((16,), jnp.float32)
        return 0

    lax.fori_loop(0, DR, zr, 0)

    for t in range(20):
        for jj in range(8):
            st = min(jj * 16, 125 - 16)
            idxbuf[t, pl.ds(st, 16)] = t * 125 + st + iota

    @pl.when(sid == 0)
    def _():
        pltpu.sync_copy(denloc, sdden)

    def issue(k, didx, lbuf, sem):
        base = ebase + k * CH
        pltpu.async_copy(dst_h.at[pl.ds(base, CH)], didx, sem)
        pltpu.async_copy(log_h.at[pl.ds(base * H, CH * H)], lbuf, sem)

    def wait(k, didx, lbuf, sem):
        base = ebase + k * CH
        pltpu.make_async_copy(dst_h.at[pl.ds(base, CH)], didx, sem).wait()
        pltpu.make_async_copy(log_h.at[pl.ds(base * H, CH * H)], lbuf,
                              sem).wait()

    def compute(k, didx, lbuf):
        base = ebase + k * CH
        for gi in range(NG):
            e16 = gi * 16 + iota
            d16 = didx[pl.ds(gi * 16, 16)]
            for h in range(H):
                l16 = plsc.load_gather(lbuf, [e16 * H + h])
                ev = jnp.exp(l16 - gbuf[pl.ds(h * 16, 16)])
                plsc.store_scatter(exbuf, [e16 * H + h], ev)
                f16 = d16 * H + h
                plsc.addupdate_scatter(
                    denloc,
                    [lax.shift_right_logical(f16, 4),
                     lax.bitwise_and(f16, 15)], ev)
        pltpu.sync_copy(exbuf, ex_h.at[pl.ds(base * H, CH * H)])

    issue(0, didx, lbuf, sem1)

    def body(m, _):
        issue(2 * m + 1, didx2, lbuf2, sem2)
        wait(2 * m, didx, lbuf, sem1)
        compute(2 * m, didx, lbuf)

        @pl.when(2 * m + 2 < NCH)
        def _():
            issue(2 * m + 2, didx, lbuf, sem1)

        wait(2 * m + 1, didx2, lbuf2, sem2)
        compute(2 * m + 1, didx2, lbuf2)
        return 0

    lax.fori_loop(0, NCH // 2, body, 0)
    plsc.subcore_barrier()
    for t in range(20):
        pltpu.sync_copy(denloc.at[pl.ds(t * 125, 125)],
                        sdden.at[idxbuf.at[t]], add=True)
    plsc.subcore_barrier()

    @pl.when(sid == 0)
    def _():
        pltpu.sync_copy(sdden, denloc)
        pltpu.sync_copy(denloc, denp_h.at[cid])


def _sc_denom(logits, dst, gmb):
    return pl.kernel(
        _denom_body,
        out_type=[
            jax.ShapeDtypeStruct((E * H,), jnp.float32),
            jax.ShapeDtypeStruct((2, DR, 16), jnp.float32),
        ],
        mesh=_mesh(),
        compiler_params=pltpu.CompilerParams(**_SC_PARAMS),
        scratch_types=[
            pltpu.VMEM((CH,), jnp.int32),
            pltpu.VMEM((CH * H,), jnp.float32),
            pltpu.VMEM((CH,), jnp.int32),
            pltpu.VMEM((CH * H,), jnp.float32),
            pltpu.VMEM((CH * H,), jnp.float32),
            pltpu.VMEM((H * 16,), jnp.float32),
            pltpu.VMEM((DR, 16), jnp.float32),
            pltpu.VMEM((20, 125), jnp.int32),
            pltpu.VMEM_SHARED((DR, 16), jnp.float32),
            pltpu.SemaphoreType.DMA,
            pltpu.SemaphoreType.DMA,
        ],
    )(logits, dst, gmb)


def _scatter_body(ex_h, rdenp_h, src_h, dst_h, xlp_h, outp_h,
                  sidx, didx, rowbuf, rdrows, exbuf,
                  sidx2, didx2, rowbuf2, rdrows2, exbuf2, albuf, msgbuf,
                  dumpbuf, sacc, sem1, sem2,
                  *, W, Wp, C, Wout, mean):
    cid = lax.axis_index("c")
    sid = lax.axis_index("s")
    wid = sid * 2 + cid
    ebase = wid * EW
    iota = lax.iota(jnp.int32, 16)
    nv = Wout // 16

    def zb(i, _):
        r = i // nv
        j = i - r * nv
        dumpbuf[r, pl.ds(j * 16, 16)] = jnp.zeros((16,), jnp.float32)
        return 0

    lax.fori_loop(0, 40 * nv, zb, 0)

    @pl.when(sid < 10)
    def _():
        for t in range(25):
            pltpu.sync_copy(dumpbuf, sacc.at[pl.ds(sid * 1000 + t * 40, 40)])

    plsc.subcore_barrier()

    scale = 1.0 / H if mean else 1.0

    def issue(k, sidx, didx, rowbuf, rdrows, exbuf, sem):
        base = ebase + k * CH
        pltpu.sync_copy(src_h.at[pl.ds(base, CH)], sidx)
        pltpu.sync_copy(dst_h.at[pl.ds(base, CH)], didx)
        pltpu.sync_copy(ex_h.at[pl.ds(base * H, CH * H)], exbuf)
        pltpu.async_copy(xlp_h.at[sidx], rowbuf, sem)
        pltpu.async_copy(rdenp_h.at[didx], rdrows, sem)

    def wait(sidx, didx, rowbuf, rdrows, exbuf, sem):
        pltpu.make_async_copy(xlp_h.at[sidx], rowbuf, sem).wait()
        pltpu.make_async_copy(rdenp_h.at[didx], rdrows, sem).wait()

    def compute(k, didx, rowbuf, rdrows, exbuf):
        for gi in range(NG):
            e16 = gi * 16 + iota
            for h in range(H):
                rd = plsc.load_gather(rdrows, [e16, jnp.full((16,), h, jnp.int32)])
                ev = plsc.load_gather(exbuf, [e16 * H + h])
                a16 = ev * rd * scale
                plsc.store_scatter(albuf, [e16 * H + h], a16)

        def ebody(e, _):
            es = jnp.full((16,), 0, jnp.int32) + e * H
            avs = [plsc.load_gather(albuf, [es + h]) for h in range(H)]
            if not mean:
                for h in range(H):
                    for jb in range(C // 32):
                        w0 = h * (C // 2) + jb * 16
                        cb = h * C + jb * 32
                        xw = rowbuf[e, pl.ds(w0, 16)]
                        msgbuf[e, pl.ds(cb, 16)] = _lo(xw) * avs[h]
                        msgbuf[e, pl.ds(cb + 16, 16)] = _hi(xw) * avs[h]
            else:
                for jb in range(C // 32):
                    acclo = jnp.zeros((16,), jnp.float32)
                    acchi = jnp.zeros((16,), jnp.float32)
                    for h in range(H):
                        xw = rowbuf[e, pl.ds(h * (C // 2) + jb * 16, 16)]
                        acclo = acclo + _lo(xw) * avs[h]
                        acchi = acchi + _hi(xw) * avs[h]
                    msgbuf[e, pl.ds(jb * 32, 16)] = acclo
                    msgbuf[e, pl.ds(jb * 32 + 16, 16)] = acchi
            return 0

        lax.fori_loop(0, CH, ebody, 0)
        pltpu.sync_copy(msgbuf, sacc.at[didx], add=True)

    bufA = (sidx, didx, rowbuf, rdrows, exbuf, sem1)
    bufB = (sidx2, didx2, rowbuf2, rdrows2, exbuf2, sem2)
    issue(0, *bufA)

    def body(m, _):
        issue(2 * m + 1, *bufB)
        wait(*bufA)
        compute(2 * m, bufA[1], bufA[2], bufA[3], bufA[4])

        @pl.when(2 * m + 2 < NCH)
        def _():
            issue(2 * m + 2, *bufA)

        wait(*bufB)
        compute(2 * m + 1, bufB[1], bufB[2], bufB[3], bufB[4])
        return 0

    lax.fori_loop(0, NCH // 2, body, 0)
    plsc.subcore_barrier()

    @pl.when(sid < 10)
    def _():
        for t in range(25):
            rbase = sid * 1000 + t * 40
            pltpu.sync_copy(sacc.at[pl.ds(rbase, 40)], dumpbuf)
            pltpu.sync_copy(dumpbuf, outp_h.at[cid, pl.ds(rbase, 40)])


def _sc_scatter(ex, rdenp, src, dst, xlp, mean):
    Wp2 = xlp.shape[1]
    W = Wp2 * 2 - 16
    C = W // H
    Wout = C if mean else W
    body = functools.partial(_scatter_body, W=W, Wp=Wp2, C=C, Wout=Wout,
                             mean=mean)
    return pl.kernel(
        body,
        out_type=jax.ShapeDtypeStruct((2, N, Wout), jnp.float32),
        mesh=_mesh(),
        compiler_params=pltpu.CompilerParams(**_SC_PARAMS),
        scratch_types=[
            pltpu.VMEM((CH,), jnp.int32),
            pltpu.VMEM((CH,), jnp.int32),
            pltpu.VMEM((CH, Wp2), jnp.int32),
            pltpu.VMEM((CH, 16), jnp.float32),
            pltpu.VMEM((CH * H,), jnp.float32),
            pltpu.VMEM((CH,), jnp.int32),
            pltpu.VMEM((CH,), jnp.int32),
            pltpu.VMEM((CH, Wp2), jnp.int32),
            pltpu.VMEM((CH, 16), jnp.float32),
            pltpu.VMEM((CH * H,), jnp.float32),
            pltpu.VMEM((CH * H,), jnp.float32),
            pltpu.VMEM((CH, Wout), jnp.float32),
            pltpu.VMEM((40, Wout), jnp.float32),
            pltpu.VMEM_SHARED((N, Wout), jnp.float32),
            pltpu.SemaphoreType.DMA,
            pltpu.SemaphoreType.DMA,
        ],
    )(ex, rdenp, src, dst, xlp)


def _attbd(att):
    heads, c = att.shape
    m = heads * c
    idx = jnp.arange(m)
    return (jnp.zeros((m, heads), jnp.float32)
            .at[idx, idx // c].set(S1 * att.reshape(-1)))


def _pack16(x):
    n, w = x.shape
    return lax.bitcast_convert_type(
        x.astype(jnp.bfloat16).reshape(n, w // 2, 2), jnp.int32)


def _deperm(outp):
    b, n, w = outp.shape
    return outp.reshape(b, n, w // 32, 2, 16).swapaxes(-1, -2).reshape(b, n, w)


def _gat_layer(xl, xr, al, ar, att, src, dst, mean):
    n, w = xl.shape
    pad = jnp.zeros((n, 12), jnp.float32)
    xlp = _pack16(jnp.concatenate([xl, al, pad], axis=1))
    xrp = _pack16(jnp.concatenate([xr, ar, pad], axis=1))
    atts2 = _pack16((S2 * att.reshape(-1)).reshape(1, w)).reshape(w // 2)
    logits, wmax = _sc_logits(xlp, xrp, src, dst, atts2)
    gm = jnp.max(wmax.reshape(NW, H, 16), axis=(0, 2))
    gmb = jnp.broadcast_to(gm[:, None], (H, 16)).reshape(H * 16)
    ex, denp = _sc_denom(logits, dst, gmb)
    rdenq = _rdenom(denp)
    rdenp = jnp.concatenate([rdenq.reshape(N, H),
                             jnp.zeros((N, 16 - H), jnp.float32)], axis=1)
    return _deperm(_sc_scatter(ex, rdenp, src, dst, xlp, mean))


def kernel(x, edge_index, Wl1, bl1, Wr1, br1, att1, bias1, g1, b1,
           Wl2, bl2, Wr2, br2, att2, bias2, g2, b2):
    src = edge_index[0]
    dst = edge_index[1]

    xl1, xr1, al1, ar1 = _proj(x, Wl1, bl1, Wr1, br1, _attbd(att1))
    outp1 = _gat_layer(xl1, xr1, al1, ar1, att1, src, dst, mean=False)
    xl2, xr2, al2, ar2 = _combine_proj(outp1, bias1, g1, b1,
                                       Wl2, bl2, Wr2, br2, _attbd(att2))
    outp2 = _gat_layer(xl2, xr2, al2, ar2, att2, src, dst, mean=True)
    return _combine_epi(outp2, bias2, g2, b2)
